# baseline (speedup 1.0000x reference)
import functools

import jax
import jax.numpy as jnp
from jax import lax
from jax.experimental import pallas as pl
from jax.experimental.pallas import tpu as pltpu
from jax.experimental.pallas import tpu_sc as plsc

NC = 2
NS = 16
NW = NC * NS
LANES = 16

BATCH = 16384
EMBED_DIM = 32
PER_W = BATCH // NW
CHUNK = 128
NCHUNK = PER_W // CHUNK

BBLOCK = 64
NBBLOCK = PER_W // BBLOCK


def _dots_kernel(user_ids, show_ids, user_emb, show_emb,
                 out, idx_u, idx_s, ue_v, se_v, out_v, sem):
    wid = lax.axis_index("c") * NS + lax.axis_index("s")
    base = wid * PER_W

    pltpu.sync_copy(user_ids.at[pl.ds(base, PER_W)], idx_u)
    pltpu.sync_copy(show_ids.at[pl.ds(base, PER_W)], idx_s)

    copies = []
    for c in range(NCHUNK):
        csl = pl.ds(c * CHUNK, CHUNK)
        copies.append(pltpu.async_copy(
            user_emb.at[idx_u.at[csl]], ue_v.at[csl], sem))
        copies.append(pltpu.async_copy(
            show_emb.at[idx_s.at[csl]], se_v.at[csl], sem))
    for cp in copies:
        cp.wait()

    iota = lax.iota(jnp.int32, LANES)
    cols = [(iota + d) % EMBED_DIM for d in range(EMBED_DIM)]

    def body(g, _):
        row = g * LANES + iota
        accs = [jnp.zeros((LANES,), jnp.float32) for _ in range(4)]
        for d in range(EMBED_DIM):
            u = plsc.load_gather(ue_v, [row, cols[d]])
            s = plsc.load_gather(se_v, [row, cols[d]])
            accs[d % 4] = accs[d % 4] + u * s
        out_v[pl.ds(g * LANES, LANES)] = (accs[0] + accs[1]) + (
            accs[2] + accs[3])
        return 0

    lax.fori_loop(0, PER_W // LANES, body, 0)

    pltpu.sync_copy(out_v, out.at[pl.ds(base, PER_W)])


def _bias_kernel(user_ids, show_ids, user_bias, show_bias,
                 out, idx_u, idx_s, ub_t, sb_t, out_v, sem):
    wid = lax.axis_index("c") * NS + lax.axis_index("s")
    base = wid * PER_W

    pltpu.sync_copy(user_ids.at[pl.ds(base, PER_W)], idx_u)
    pltpu.sync_copy(show_ids.at[pl.ds(base, PER_W)], idx_s)

    iota = lax.iota(jnp.int32, LANES)
    zeros = jnp.zeros((LANES,), jnp.int32)

    def block_body(b, _):
        off = b * BBLOCK
        for k0 in range(0, BBLOCK, LANES):
            uids = idx_u[pl.ds(off + k0, LANES)]
            sids = idx_s[pl.ds(off + k0, LANES)]
            for k in range(LANES):
                pltpu.async_copy(user_bias.at[uids[k]], ub_t.at[k0 + k], sem)
                pltpu.async_copy(show_bias.at[sids[k]], sb_t.at[k0 + k], sem)
        pltpu.make_async_copy(user_bias.at[pl.ds(0, BBLOCK)], ub_t,
                              sem).wait()
        pltpu.make_async_copy(show_bias.at[pl.ds(0, BBLOCK)], sb_t,
                              sem).wait()
        for g in range(BBLOCK // LANES):
            row = g * LANES + iota
            ub = plsc.load_gather(ub_t, [row, zeros])
            sb = plsc.load_gather(sb_t, [row, zeros])
            out_v[pl.ds(off + g * LANES, LANES)] = ub + sb
        return 0

    lax.fori_loop(0, NBBLOCK, block_body, 0)

    pltpu.sync_copy(out_v, out.at[pl.ds(base, PER_W)])


def _add_kernel(dots, biases, out, a_v, b_v, o_v):
    wid = lax.axis_index("c") * NS + lax.axis_index("s")
    base = wid * PER_W
    pltpu.sync_copy(dots.at[pl.ds(base, PER_W)], a_v)
    pltpu.sync_copy(biases.at[pl.ds(base, PER_W)], b_v)
    for j in range(PER_W // LANES):
        sl = pl.ds(j * LANES, LANES)
        o_v[sl] = a_v[sl] + b_v[sl]
    pltpu.sync_copy(o_v, out.at[pl.ds(base, PER_W)])


def _mesh():
    return plsc.VectorSubcoreMesh(
        core_axis_name="c", subcore_axis_name="s",
        num_cores=NC, num_subcores=NS)


@jax.jit
def _mf(user_ids, show_ids, user_emb, show_emb, user_bias, show_bias):
    dots_fn = pl.kernel(
        _dots_kernel,
        out_type=jax.ShapeDtypeStruct((BATCH,), jnp.float32),
        mesh=_mesh(),
        scratch_types=[
            pltpu.VMEM((PER_W,), jnp.int32),
            pltpu.VMEM((PER_W,), jnp.int32),
            pltpu.VMEM((PER_W, EMBED_DIM), jnp.float32),
            pltpu.VMEM((PER_W, EMBED_DIM), jnp.float32),
            pltpu.VMEM((PER_W,), jnp.float32),
            pltpu.SemaphoreType.DMA,
        ],
        compiler_params=pltpu.CompilerParams(
            needs_layout_passes=False, use_tc_tiling_on_sc=False),
    )
    bias_fn = pl.kernel(
        _bias_kernel,
        out_type=jax.ShapeDtypeStruct((BATCH,), jnp.float32),
        mesh=_mesh(),
        scratch_types=[
            pltpu.VMEM((PER_W,), jnp.int32),
            pltpu.VMEM((PER_W,), jnp.int32),
            pltpu.VMEM((BBLOCK, 1), jnp.float32),
            pltpu.VMEM((BBLOCK, 1), jnp.float32),
            pltpu.VMEM((PER_W,), jnp.float32),
            pltpu.SemaphoreType.DMA,
        ],
        compiler_params=pltpu.CompilerParams(
            needs_layout_passes=False, use_tc_tiling_on_sc=True),
    )
    add_fn = pl.kernel(
        _add_kernel,
        out_type=jax.ShapeDtypeStruct((BATCH,), jnp.float32),
        mesh=_mesh(),
        scratch_types=[
            pltpu.VMEM((PER_W,), jnp.float32),
            pltpu.VMEM((PER_W,), jnp.float32),
            pltpu.VMEM((PER_W,), jnp.float32),
        ],
        compiler_params=pltpu.CompilerParams(
            needs_layout_passes=False, use_tc_tiling_on_sc=True),
    )
    dots = dots_fn(user_ids, show_ids, user_emb, show_emb)
    biases = bias_fn(user_ids, show_ids, user_bias, show_bias)
    return add_fn(dots, biases)


def kernel(user_ids, show_ids, user_emb, show_emb, user_bias, show_bias):
    return _mf(user_ids.astype(jnp.int32), show_ids.astype(jnp.int32),
               user_emb, show_emb, user_bias, show_bias)


# speedup vs baseline: 1.3314x; 1.3314x over previous
import functools

import jax
import jax.numpy as jnp
from jax import lax
from jax.experimental import pallas as pl
from jax.experimental.pallas import tpu as pltpu
from jax.experimental.pallas import tpu_sc as plsc

NC = 2
NS = 16
NW = NC * NS
LANES = 16

BATCH = 16384
EMBED_DIM = 32
PER_W = BATCH // NW
CHUNK = 128
NCHUNK = PER_W // CHUNK


def _mf_kernel(user_ids, show_ids, user_emb, show_emb, user_bias, show_bias,
               out, idx_u, idx_s, ue_v, se_v, ub_v, sb_v, out_v, sem):
    wid = lax.axis_index("c") * NS + lax.axis_index("s")
    base = wid * PER_W

    pltpu.sync_copy(user_ids.at[pl.ds(base, PER_W)], idx_u)
    pltpu.sync_copy(show_ids.at[pl.ds(base, PER_W)], idx_s)

    copies = []
    for c in range(NCHUNK):
        iu = idx_u.at[pl.ds(c * CHUNK, CHUNK)]
        isw = idx_s.at[pl.ds(c * CHUNK, CHUNK)]
        rows = pl.ds(c * CHUNK, CHUNK)
        copies.append(pltpu.async_copy(user_emb.at[iu], ue_v.at[rows], sem))
        copies.append(pltpu.async_copy(show_emb.at[isw], se_v.at[rows], sem))
        copies.append(pltpu.async_copy(user_bias.at[iu], ub_v.at[rows], sem))
        copies.append(pltpu.async_copy(show_bias.at[isw], sb_v.at[rows], sem))
    for cp in copies:
        cp.wait()

    iota = lax.iota(jnp.int32, LANES)
    cols = [(iota + d) % EMBED_DIM for d in range(EMBED_DIM)]

    def body(g, _):
        row = g * LANES + iota
        accs = [jnp.zeros((LANES,), jnp.float32) for _ in range(4)]
        for d in range(EMBED_DIM):
            u = plsc.load_gather(ue_v, [row, cols[d]])
            s = plsc.load_gather(se_v, [row, cols[d]])
            accs[d % 4] = accs[d % 4] + u * s
        ub = ub_v[pl.ds(g * LANES, LANES)]
        sb = sb_v[pl.ds(g * LANES, LANES)]
        res = (accs[0] + accs[1]) + (accs[2] + accs[3]) + (ub + sb)
        out_v[pl.ds(g * LANES, LANES)] = res
        return 0

    lax.fori_loop(0, PER_W // LANES, body, 0)

    pltpu.sync_copy(out_v, out.at[pl.ds(base, PER_W)])


@jax.jit
def _mf(user_ids, show_ids, user_emb, show_emb, user_bias, show_bias):
    mesh = plsc.VectorSubcoreMesh(
        core_axis_name="c", subcore_axis_name="s",
        num_cores=NC, num_subcores=NS)
    fn = pl.kernel(
        _mf_kernel,
        out_type=jax.ShapeDtypeStruct((BATCH,), jnp.float32),
        mesh=mesh,
        scratch_types=[
            pltpu.VMEM((PER_W,), jnp.int32),
            pltpu.VMEM((PER_W,), jnp.int32),
            pltpu.VMEM((PER_W, EMBED_DIM), jnp.float32),
            pltpu.VMEM((PER_W, EMBED_DIM), jnp.float32),
            pltpu.VMEM((PER_W,), jnp.float32),
            pltpu.VMEM((PER_W,), jnp.float32),
            pltpu.VMEM((PER_W,), jnp.float32),
            pltpu.SemaphoreType.DMA,
        ],
        compiler_params=pltpu.CompilerParams(
            needs_layout_passes=False, use_tc_tiling_on_sc=False),
    )
    return fn(user_ids, show_ids, user_emb, show_emb, user_bias, show_bias)


def kernel(user_ids, show_ids, user_emb, show_emb, user_bias, show_bias):
    return _mf(user_ids.astype(jnp.int32), show_ids.astype(jnp.int32),
               user_emb, show_emb,
               user_bias.reshape(-1), show_bias.reshape(-1))
